# baseline (speedup 1.0000x reference)
import jax
import jax.numpy as jnp
from jax.experimental import pallas as pl

N_IN = 128
EMB_DIM = 512
BPG = 8


def _embed_kernel(inp_ref, topo_ref, out_ref):
    G = topo_ref.shape[0]
    for j in range(BPG):
        out_ref[pl.ds(j * G, G), 0, :N_IN] = inp_ref[j]
        out_ref[pl.ds(j * G, G), 0, N_IN:] = topo_ref[:, : EMB_DIM - N_IN]


def kernel(inputs, grid_positions, embedding, topographical_embedding, x_learn, y_learn):
    B, GRID, _ = inputs.shape

    out = pl.pallas_call(
        _embed_kernel,
        grid=(B // BPG,),
        in_specs=[
            pl.BlockSpec((BPG, GRID, N_IN), lambda c: (c, 0, 0)),
            pl.BlockSpec((GRID, EMB_DIM), lambda c: (0, 0)),
        ],
        out_specs=pl.BlockSpec((BPG * GRID, 1, EMB_DIM), lambda c: (c, 0, 0)),
        out_shape=jax.ShapeDtypeStruct((B * GRID, 1, EMB_DIM), jnp.float32),
    )(inputs, topographical_embedding)
    return out
